# baseline (speedup 1.0000x reference)
import jax
import jax.numpy as jnp
from jax import lax
from jax.experimental import pallas as pl
from jax.experimental.pallas import tpu as pltpu
from jax.experimental.pallas import tpu_sc as plsc

B = 16384
D = 32
N = 1000000
RPB = 4
W = 128
NC = 2
NS = 16
NW = NC * NS
BPW = B // NW
GRP = 16
ICH = 128
NCH = BPW // ICH
SROWS = 256
NSTG = BPW // SROWS
GPS = SROWS // GRP


def _cmf_body(users_hbm, items_hbm, uembr_hbm, iembr_hbm, out_hbm,
              uidx_v, iidx_v,
              ublk0, ublk1, ublk2, ublk3,
              iblk0, iblk1, iblk2, iblk3,
              urows_v, irows_v, out_v, sem):
    wid = lax.axis_index("s") * NC + lax.axis_index("c")
    base = wid * BPW

    pltpu.sync_copy(users_hbm.at[pl.ds(base, BPW)], uidx_v)
    pltpu.sync_copy(items_hbm.at[pl.ds(base, BPW)], iidx_v)

    ublks = [ublk0, ublk1, ublk2, ublk3]
    iblks = [iblk0, iblk1, iblk2, iblk3]
    for j in range(NCH):
        for k in range(ICH // GRP):
            src = pl.ds(j * ICH + k * GRP, GRP)
            dst = pl.ds(k * GRP, GRP)
            ublks[j][dst] = uidx_v[src] >> 2
            iblks[j][dst] = iidx_v[src] >> 2

    lane = lax.iota(jnp.int32, GRP)

    for s in range(NSTG):
        copies = []
        for c in range(SROWS // ICH):
            j = s * (SROWS // ICH) + c
            copies.append(pltpu.async_copy(
                uembr_hbm.at[ublks[j]],
                urows_v.at[pl.ds(c * ICH, ICH)], sem))
            copies.append(pltpu.async_copy(
                iembr_hbm.at[iblks[j]],
                irows_v.at[pl.ds(c * ICH, ICH)], sem))
        for cp in copies:
            cp.wait()

        def group(g, carry):
            row = g * GRP + lane
            uvec = uidx_v[pl.ds(s * SROWS + g * GRP, GRP)]
            ivec = iidx_v[pl.ds(s * SROWS + g * GRP, GRP)]
            uoff = (uvec & (RPB - 1)) * D
            ioff = (ivec & (RPB - 1)) * D
            acc = (plsc.load_gather(urows_v, [row, uoff + lane])
                   * plsc.load_gather(irows_v, [row, ioff + lane]))
            for t in range(1, D):
                dcol = (lane + t) & (D - 1)
                acc = acc + (plsc.load_gather(urows_v, [row, uoff + dcol])
                             * plsc.load_gather(irows_v, [row, ioff + dcol]))
            out_v[pl.ds(s * SROWS + g * GRP, GRP)] = acc
            return carry

        lax.fori_loop(0, GPS, group, 0)

    pltpu.sync_copy(out_v, out_hbm.at[pl.ds(base, BPW)])


@jax.jit
def kernel(users, items, user_emb, item_emb):
    users1 = users.astype(jnp.int32)
    items1 = items.astype(jnp.int32)
    uembr = user_emb.reshape(N // RPB, W)
    iembr = item_emb.reshape(N // RPB, W)
    mesh = plsc.VectorSubcoreMesh(core_axis_name="c", subcore_axis_name="s")
    run = pl.kernel(
        _cmf_body,
        out_type=jax.ShapeDtypeStruct((B,), jnp.float32),
        mesh=mesh,
        compiler_params=pltpu.CompilerParams(needs_layout_passes=False),
        scratch_types=[
            pltpu.VMEM((BPW,), jnp.int32),
            pltpu.VMEM((BPW,), jnp.int32),
            pltpu.VMEM((ICH,), jnp.int32),
            pltpu.VMEM((ICH,), jnp.int32),
            pltpu.VMEM((ICH,), jnp.int32),
            pltpu.VMEM((ICH,), jnp.int32),
            pltpu.VMEM((ICH,), jnp.int32),
            pltpu.VMEM((ICH,), jnp.int32),
            pltpu.VMEM((ICH,), jnp.int32),
            pltpu.VMEM((ICH,), jnp.int32),
            pltpu.VMEM((SROWS, W), jnp.float32),
            pltpu.VMEM((SROWS, W), jnp.float32),
            pltpu.VMEM((BPW,), jnp.float32),
            pltpu.SemaphoreType.DMA,
        ],
    )
    return run(users1, items1, uembr, iembr)
